# baseline (speedup 1.0000x reference)
import jax
import jax.numpy as jnp
from jax import lax
from jax.experimental import pallas as pl
from jax.experimental.pallas import tpu as pltpu
from jax.experimental.pallas import tpu_sc as plsc

N = 10000
D = 128
E = 320000
H = 5000
P = 160000

NC = 2
NS = 16
NW = NC * NS
CHUNK = 128
NBUF = 3
G = 42
RND_A = 2
RND_H1 = 2
RND_H2 = 1
CH_A = G * RND_A * NW
CH_P = G * RND_H1 * NS
ZROWS = 25
WROWS = 40


def _cdiv(a, b):
    return (a + b - 1) // b


def _sc_body(ego_bf, a_rows, a_cols, a_vals,
             p1r, p1c, p1v, p2v,
             l1r, l1c, l1v, l2v,
             out0, out1,
             idx_d, idx_s, vb0, vb1, vb2, gb0, gb1, gb2, sb0, sb1, sb2,
             zbuf_bf,
             gs0, gs1, gs2, ss0, ss1, ss2, vs0, vs1, vs2,
             acc_sh, h_sh):
    c = lax.axis_index("c")
    s = lax.axis_index("s")
    w32 = s * NC + c
    vbufs = (vb0, vb1, vb2)
    gbufs = (gb0, gb1, gb2)
    sbufs = (sb0, sb1, sb2)
    gs = (gs0, gs1, gs2)
    ss = (ss0, ss1, ss2)
    vs = (vs0, vs1, vs2)

    zero32bf = jnp.zeros((32,), jnp.bfloat16)

    @pl.loop(0, ZROWS)
    def _(r):
        for j in range(D // 32):
            zbuf_bf[r, pl.ds(j * 32, 32)] = zero32bf

    n_chunks_acc = N // ZROWS
    n_chunks_h = H // ZROWS

    @pl.loop(0, n_chunks_acc // NS)
    def _(it):
        ch = it * NS + s
        pltpu.sync_copy(zbuf_bf, acc_sh.at[pl.ds(ch * ZROWS, ZROWS)])

    def zero_h():
        @pl.loop(0, _cdiv(n_chunks_h, NS))
        def _(it):
            ch = it * NS + s

            @pl.when(ch < n_chunks_h)
            def _():
                pltpu.sync_copy(zbuf_bf, h_sh.at[pl.ds(ch * ZROWS, ZROWS)])

    zero_h()
    plsc.subcore_barrier()

    def scale(gbuf, vbuf, sbuf):
        @pl.loop(0, CHUNK // 8)
        def _(grp):
            for k in range(8):
                i = grp * 8 + k
                sval = vbuf[i]
                for j in range(D // 32):
                    sl = pl.ds(j * 32, 32)
                    sbuf[i, sl] = gbuf[i, sl] * sval

    def process(src_ref, dst_sh, rows_h, cols_h, vals_h, n_rounds, wid):
        q_iters = G // NBUF

        def issue_gather(b, g, base):
            pltpu.async_copy(src_ref.at[idx_s.at[g]], gbufs[b], gs[b])
            pltpu.async_copy(vals_h.at[pl.ds((base + g) * CHUNK, CHUNK)],
                             vbufs[b], vs[b])

        def wait_gather(b, g, base):
            pltpu.make_async_copy(src_ref.at[idx_s.at[g]],
                                  gbufs[b], gs[b]).wait()
            pltpu.make_async_copy(vals_h.at[pl.ds(0, CHUNK)],
                                  vbufs[b], vs[b]).wait()

        @pl.loop(0, n_rounds)
        def _(r):
            base = wid * (n_rounds * G) + r * G
            pltpu.sync_copy(rows_h.at[pl.ds(base, G)], idx_d)
            pltpu.sync_copy(cols_h.at[pl.ds(base, G)], idx_s)

            for b in range(NBUF):
                issue_gather(b, b, base)

            @pl.loop(0, q_iters)
            def _(q):
                for b in range(NBUF):
                    g = q * NBUF + b
                    wait_gather(b, g, base)

                    @pl.when(q > 0)
                    def _(b=b):
                        pltpu.make_async_copy(sbufs[b],
                                              dst_sh.at[idx_d.at[0]],
                                              ss[b]).wait()

                    scale(gbufs[b], vbufs[b], sbufs[b])
                    pltpu.async_copy(sbufs[b], dst_sh.at[idx_d.at[g]], ss[b],
                                     add=True)

                    @pl.when(q < q_iters - 1)
                    def _(b=b, g=g):
                        issue_gather(b, g + NBUF, base)

            for b in range(NBUF):
                pltpu.make_async_copy(sbufs[b], dst_sh.at[idx_d.at[0]],
                                      ss[b]).wait()

    process(ego_bf, acc_sh, a_rows, a_cols, a_vals, RND_A, w32)

    process(ego_bf, h_sh, p1r, p1c, p1v, RND_H1, s)
    plsc.subcore_barrier()

    process(h_sh, acc_sh, p1c, p1r, p2v, RND_H2, w32)
    plsc.subcore_barrier()

    zero_h()
    plsc.subcore_barrier()
    process(ego_bf, h_sh, l1r, l1c, l1v, RND_H1, s)
    plsc.subcore_barrier()
    process(h_sh, acc_sh, l1c, l1r, l2v, RND_H2, w32)
    plsc.subcore_barrier()

    @pl.loop(0, _cdiv(N // WROWS, NS))
    def _(it):
        ch = it * NS + s

        @pl.when(ch < N // WROWS)
        def _():
            sl = pl.ds(ch * WROWS, WROWS)

            @pl.when(c == 0)
            def _():
                pltpu.sync_copy(acc_sh.at[sl], out0.at[sl])

            @pl.when(c == 1)
            def _():
                pltpu.sync_copy(acc_sh.at[sl], out1.at[sl])


def _sc_aggregate(ego_bf, a_rows, a_cols, a_vals, p1r, p1c, p1v, p2v,
                  l1r, l1c, l1v, l2v):
    mesh = plsc.VectorSubcoreMesh(core_axis_name="c", subcore_axis_name="s")
    f = pl.kernel(
        _sc_body,
        out_type=(
            jax.ShapeDtypeStruct((N, D), jnp.bfloat16),
            jax.ShapeDtypeStruct((N, D), jnp.bfloat16),
        ),
        mesh=mesh,
        compiler_params=pltpu.CompilerParams(use_tc_tiling_on_sc=False),
        scratch_types=(
            pltpu.VMEM((G, CHUNK), jnp.int32),
            pltpu.VMEM((G, CHUNK), jnp.int32),
            pltpu.VMEM((CHUNK, 32), jnp.bfloat16),
            pltpu.VMEM((CHUNK, 32), jnp.bfloat16),
            pltpu.VMEM((CHUNK, 32), jnp.bfloat16),
            pltpu.VMEM((CHUNK, D), jnp.bfloat16),
            pltpu.VMEM((CHUNK, D), jnp.bfloat16),
            pltpu.VMEM((CHUNK, D), jnp.bfloat16),
            pltpu.VMEM((CHUNK, D), jnp.bfloat16),
            pltpu.VMEM((CHUNK, D), jnp.bfloat16),
            pltpu.VMEM((CHUNK, D), jnp.bfloat16),
            pltpu.VMEM((ZROWS, D), jnp.bfloat16),
            pltpu.SemaphoreType.DMA,
            pltpu.SemaphoreType.DMA,
            pltpu.SemaphoreType.DMA,
            pltpu.SemaphoreType.DMA,
            pltpu.SemaphoreType.DMA,
            pltpu.SemaphoreType.DMA,
            pltpu.SemaphoreType.DMA,
            pltpu.SemaphoreType.DMA,
            pltpu.SemaphoreType.DMA,
            pltpu.VMEM_SHARED((N, D), jnp.bfloat16),
            pltpu.VMEM_SHARED((H, D), jnp.bfloat16),
        ),
    )
    return f(ego_bf, a_rows, a_cols, a_vals, p1r, p1c, p1v, p2v,
             l1r, l1c, l1v, l2v)


BM = 1000


def _mlp_body(ego_ref, p0_ref, p1_ref, w1_ref, b1_ref, w2_ref, b2_ref,
              o_ref):
    ego = ego_ref[...]
    side = (p0_ref[...].astype(jnp.float32)
            + p1_ref[...].astype(jnp.float32))
    dn = (((1,), (1,)), ((), ()))
    x1 = lax.dot_general(ego + side, w1_ref[...], dn,
                         preferred_element_type=jnp.float32) + b1_ref[...]
    x2 = lax.dot_general(ego * side, w2_ref[...], dn,
                         preferred_element_type=jnp.float32) + b2_ref[...]
    o_ref[...] = (jnp.where(x1 > 0, x1, 0.01 * x1)
                  + jnp.where(x2 > 0, x2, 0.01 * x2))


def _mlp(ego, part0, part1, W1, b1, W2, b2):
    grid = (N // BM,)
    row_spec = pl.BlockSpec((BM, D), lambda i: (i, 0))
    full_spec = pl.BlockSpec((D, D), lambda i: (0, 0))
    bias_spec = pl.BlockSpec((1, D), lambda i: (0, 0))
    return pl.pallas_call(
        _mlp_body,
        grid=grid,
        in_specs=[row_spec, row_spec, row_spec,
                  full_spec, bias_spec, full_spec, bias_spec],
        out_specs=row_spec,
        out_shape=jax.ShapeDtypeStruct((N, D), jnp.float32),
    )(ego, part0, part1, W1, b1.reshape(1, D), W2, b2.reshape(1, D))


def _pad2(x, ch):
    total = ch * CHUNK
    return jnp.pad(x, (0, total - x.shape[0])).reshape(ch, CHUNK)


def _padv(x, ch):
    total = ch * CHUNK
    v = jnp.pad(x, (0, total - x.shape[0])).astype(jnp.bfloat16)
    return jnp.broadcast_to(v[:, None], (total, 32))


def kernel(ego_embeddings, A_rows, A_cols, A_vals,
           proj1_rows, proj1_cols, proj1_vals, proj2_vals,
           lib1_rows, lib1_cols, lib1_vals, lib2_vals,
           W1, b1, W2, b2):
    i32 = jnp.int32
    ego_bf = ego_embeddings.astype(jnp.bfloat16)
    part0, part1 = _sc_aggregate(
        ego_bf,
        _pad2(A_rows.astype(i32), CH_A),
        _pad2(A_cols.astype(i32), CH_A),
        _padv(A_vals, CH_A),
        _pad2(proj1_rows.astype(i32), CH_P),
        _pad2(proj1_cols.astype(i32), CH_P),
        _padv(proj1_vals, CH_P),
        _padv(proj2_vals, CH_P),
        _pad2(lib1_rows.astype(i32), CH_P),
        _pad2(lib1_cols.astype(i32), CH_P),
        _padv(lib1_vals, CH_P),
        _padv(lib2_vals, CH_P))
    return _mlp(ego_embeddings, part0, part1, W1, b1, W2, b2)


# speedup vs baseline: 1.5236x; 1.5236x over previous
import jax
import jax.numpy as jnp
from jax import lax
from jax.experimental import pallas as pl
from jax.experimental.pallas import tpu as pltpu
from jax.experimental.pallas import tpu_sc as plsc

N = 10000
D = 128
E = 320000
H = 5000
P = 160000

NC = 2
NS = 16
NW = NC * NS
CHUNK = 128
NBUF = 2
G = 40
RND_A = 2
RND_H1 = 2
RND_H2 = 1
CH_A = G * RND_A * NW
CH_P = G * RND_H1 * NS
ZROWS = 25
WROWS = 40


def _cdiv(a, b):
    return (a + b - 1) // b


def _sc_body(ego_bf, a_rows, a_cols, a_vals,
             p1r, p1c, p1v, p2v,
             l1r, l1c, l1v, l2v,
             out0, out1,
             idx_d, idx_s, vb0, vb1, gb0, gb1, sb0, sb1,
             zbuf_bf,
             gs0, gs1, ss0, ss1, vs0, vs1,
             acc_sh, h_sh):
    c = lax.axis_index("c")
    s = lax.axis_index("s")
    w32 = s * NC + c
    vbufs = (vb0, vb1)
    gbufs = (gb0, gb1)
    sbufs = (sb0, sb1)
    gs = (gs0, gs1)
    ss = (ss0, ss1)
    vs = (vs0, vs1)

    zero32bf = jnp.zeros((32,), jnp.bfloat16)

    @pl.loop(0, ZROWS)
    def _(r):
        for j in range(D // 32):
            zbuf_bf[r, pl.ds(j * 32, 32)] = zero32bf

    n_chunks_acc = N // ZROWS
    n_chunks_h = H // ZROWS

    @pl.loop(0, n_chunks_acc // NS)
    def _(it):
        ch = it * NS + s
        pltpu.sync_copy(zbuf_bf, acc_sh.at[pl.ds(ch * ZROWS, ZROWS)])

    def zero_h():
        @pl.loop(0, _cdiv(n_chunks_h, NS))
        def _(it):
            ch = it * NS + s

            @pl.when(ch < n_chunks_h)
            def _():
                pltpu.sync_copy(zbuf_bf, h_sh.at[pl.ds(ch * ZROWS, ZROWS)])

    zero_h()
    plsc.subcore_barrier()

    def scale(gbuf, vbuf, sbuf):
        @pl.loop(0, CHUNK // 8)
        def _(grp):
            for k in range(8):
                i = grp * 8 + k
                sval = vbuf[i]
                for j in range(D // 32):
                    sl = pl.ds(j * 32, 32)
                    sbuf[i, sl] = gbuf[i, sl] * sval

    def process(src_ref, dst_sh, rows_h, cols_h, vals_h, n_rounds, wid):
        q_iters = G // NBUF

        def issue_gather(b, g, base):
            pltpu.async_copy(src_ref.at[idx_s.at[g]], gbufs[b], gs[b])
            pltpu.async_copy(vals_h.at[pl.ds((base + g) * CHUNK, CHUNK)],
                             vbufs[b], vs[b])

        def wait_gather(b, g, base):
            pltpu.make_async_copy(src_ref.at[idx_s.at[g]],
                                  gbufs[b], gs[b]).wait()
            pltpu.make_async_copy(vals_h.at[pl.ds(0, CHUNK)],
                                  vbufs[b], vs[b]).wait()

        @pl.loop(0, n_rounds)
        def _(r):
            base = wid * (n_rounds * G) + r * G
            pltpu.sync_copy(rows_h.at[pl.ds(base, G)], idx_d)
            pltpu.sync_copy(cols_h.at[pl.ds(base, G)], idx_s)

            for b in range(NBUF):
                issue_gather(b, b, base)

            @pl.loop(0, q_iters)
            def _(q):
                for b in range(NBUF):
                    g = q * NBUF + b
                    wait_gather(b, g, base)

                    @pl.when(q > 0)
                    def _(b=b):
                        pltpu.make_async_copy(sbufs[b],
                                              dst_sh.at[idx_d.at[0]],
                                              ss[b]).wait()

                    scale(gbufs[b], vbufs[b], sbufs[b])
                    pltpu.async_copy(sbufs[b], dst_sh.at[idx_d.at[g]], ss[b],
                                     add=True)

                    @pl.when(q < q_iters - 1)
                    def _(b=b, g=g):
                        issue_gather(b, g + NBUF, base)

            for b in range(NBUF):
                pltpu.make_async_copy(sbufs[b], dst_sh.at[idx_d.at[0]],
                                      ss[b]).wait()

    process(ego_bf, acc_sh, a_rows, a_cols, a_vals, RND_A, w32)

    process(ego_bf, h_sh, p1r, p1c, p1v, RND_H1, s)
    plsc.subcore_barrier()

    process(h_sh, acc_sh, p1c, p1r, p2v, RND_H2, w32)
    plsc.subcore_barrier()

    zero_h()
    plsc.subcore_barrier()
    process(ego_bf, h_sh, l1r, l1c, l1v, RND_H1, s)
    plsc.subcore_barrier()
    process(h_sh, acc_sh, l1c, l1r, l2v, RND_H2, w32)
    plsc.subcore_barrier()

    @pl.loop(0, _cdiv(N // WROWS, NS))
    def _(it):
        ch = it * NS + s

        @pl.when(ch < N // WROWS)
        def _():
            sl = pl.ds(ch * WROWS, WROWS)

            @pl.when(c == 0)
            def _():
                pltpu.sync_copy(acc_sh.at[sl], out0.at[sl])

            @pl.when(c == 1)
            def _():
                pltpu.sync_copy(acc_sh.at[sl], out1.at[sl])


def _sc_aggregate(ego_bf, a_rows, a_cols, a_vals, p1r, p1c, p1v, p2v,
                  l1r, l1c, l1v, l2v):
    mesh = plsc.VectorSubcoreMesh(core_axis_name="c", subcore_axis_name="s")
    f = pl.kernel(
        _sc_body,
        out_type=(
            jax.ShapeDtypeStruct((N, D), jnp.bfloat16),
            jax.ShapeDtypeStruct((N, D), jnp.bfloat16),
        ),
        mesh=mesh,
        compiler_params=pltpu.CompilerParams(use_tc_tiling_on_sc=False),
        scratch_types=(
            pltpu.VMEM((G, CHUNK), jnp.int32),
            pltpu.VMEM((G, CHUNK), jnp.int32),
            pltpu.VMEM((CHUNK, 32), jnp.bfloat16),
            pltpu.VMEM((CHUNK, 32), jnp.bfloat16),
            pltpu.VMEM((CHUNK, D), jnp.bfloat16),
            pltpu.VMEM((CHUNK, D), jnp.bfloat16),
            pltpu.VMEM((CHUNK, D), jnp.bfloat16),
            pltpu.VMEM((CHUNK, D), jnp.bfloat16),
            pltpu.VMEM((ZROWS, D), jnp.bfloat16),
            pltpu.SemaphoreType.DMA,
            pltpu.SemaphoreType.DMA,
            pltpu.SemaphoreType.DMA,
            pltpu.SemaphoreType.DMA,
            pltpu.SemaphoreType.DMA,
            pltpu.SemaphoreType.DMA,
            pltpu.VMEM_SHARED((N, D), jnp.bfloat16),
            pltpu.VMEM_SHARED((H, D), jnp.bfloat16),
        ),
    )
    return f(ego_bf, a_rows, a_cols, a_vals, p1r, p1c, p1v, p2v,
             l1r, l1c, l1v, l2v)


BM = 1000


def _mlp_body(ego_ref, p0_ref, p1_ref, w1_ref, b1_ref, w2_ref, b2_ref,
              o_ref):
    ego = ego_ref[...]
    side = (p0_ref[...].astype(jnp.float32)
            + p1_ref[...].astype(jnp.float32))
    dn = (((1,), (1,)), ((), ()))
    x1 = lax.dot_general(ego + side, w1_ref[...], dn,
                         preferred_element_type=jnp.float32) + b1_ref[...]
    x2 = lax.dot_general(ego * side, w2_ref[...], dn,
                         preferred_element_type=jnp.float32) + b2_ref[...]
    o_ref[...] = (jnp.where(x1 > 0, x1, 0.01 * x1)
                  + jnp.where(x2 > 0, x2, 0.01 * x2))


def _mlp(ego, part0, part1, W1, b1, W2, b2):
    grid = (N // BM,)
    row_spec = pl.BlockSpec((BM, D), lambda i: (i, 0))
    full_spec = pl.BlockSpec((D, D), lambda i: (0, 0))
    bias_spec = pl.BlockSpec((1, D), lambda i: (0, 0))
    return pl.pallas_call(
        _mlp_body,
        grid=grid,
        in_specs=[row_spec, row_spec, row_spec,
                  full_spec, bias_spec, full_spec, bias_spec],
        out_specs=row_spec,
        out_shape=jax.ShapeDtypeStruct((N, D), jnp.float32),
    )(ego, part0, part1, W1, b1.reshape(1, D), W2, b2.reshape(1, D))


def _pad2(x, ch):
    total = ch * CHUNK
    return jnp.pad(x, (0, total - x.shape[0])).reshape(ch, CHUNK)


def _padv(x, ch):
    total = ch * CHUNK
    v = jnp.pad(x, (0, total - x.shape[0])).astype(jnp.bfloat16)
    return jnp.broadcast_to(v[:, None], (total, 32))


def kernel(ego_embeddings, A_rows, A_cols, A_vals,
           proj1_rows, proj1_cols, proj1_vals, proj2_vals,
           lib1_rows, lib1_cols, lib1_vals, lib2_vals,
           W1, b1, W2, b2):
    i32 = jnp.int32
    ego_bf = ego_embeddings.astype(jnp.bfloat16)
    part0, part1 = _sc_aggregate(
        ego_bf,
        _pad2(A_rows.astype(i32), CH_A),
        _pad2(A_cols.astype(i32), CH_A),
        _padv(A_vals, CH_A),
        _pad2(proj1_rows.astype(i32), CH_P),
        _pad2(proj1_cols.astype(i32), CH_P),
        _padv(proj1_vals, CH_P),
        _padv(proj2_vals, CH_P),
        _pad2(lib1_rows.astype(i32), CH_P),
        _pad2(lib1_cols.astype(i32), CH_P),
        _padv(lib1_vals, CH_P),
        _padv(lib2_vals, CH_P))
    return _mlp(ego_embeddings, part0, part1, W1, b1, W2, b2)
